# baseline (speedup 1.0000x reference)
import functools

import jax
import jax.numpy as jnp
from jax import lax
from jax.experimental import pallas as pl
from jax.experimental.pallas import tpu as pltpu
from jax.experimental.pallas import tpu_sc as plsc

N = 10000
E = 320000
D = 128
NPAD = 10240

NC = 2
NS = 16
NW = NC * NS
C = 128
EPT = E // NW
CF = EPT // C
TAIL = EPT - CF * C
RPT = NPAD // NS

_mesh = plsc.VectorSubcoreMesh(
    core_axis_name="c", subcore_axis_name="s", num_cores=NC, num_subcores=NS)


@functools.partial(
    pl.kernel,
    out_type=(jax.ShapeDtypeStruct((NPAD,), jnp.float32),
              jax.ShapeDtypeStruct((NPAD,), jnp.float32)),
    mesh=_mesh,
    scratch_types=[
        pltpu.VMEM((C,), jnp.int32),
        pltpu.VMEM((C,), jnp.int32),
        pltpu.VMEM((C,), jnp.int32),
        pltpu.VMEM((C,), jnp.int32),
        pltpu.VMEM((TAIL,), jnp.int32),
        pltpu.VMEM((C,), jnp.float32),
        pltpu.VMEM((RPT,), jnp.float32),
        pltpu.VMEM_SHARED((NPAD,), jnp.float32),
        pltpu.SemaphoreType.DMA,
        pltpu.SemaphoreType.DMA,
        pltpu.SemaphoreType.DMA,
        pltpu.SemaphoreType.DMA,
    ],
)
def _deg_call(edge_hbm, out0, out1, d0, d1, d2, d3, dt, ones_v, zv, acc,
              ds0, ds1, ds2, ds3):
    cid = lax.axis_index("c")
    sid = lax.axis_index("s")
    wid = cid * NS + sid
    ebase = E + wid * EPT

    for k in range(RPT // 16):
        zv[pl.ds(k * 16, 16)] = jnp.zeros((16,), jnp.float32)
    for k in range(C // 16):
        ones_v[pl.ds(k * 16, 16)] = jnp.ones((16,), jnp.float32)

    rb = sid * RPT
    pltpu.sync_copy(zv, acc.at[pl.ds(rb, RPT)])
    plsc.subcore_barrier()

    def d_copy(j, buf, sem):
        return pltpu.make_async_copy(
            edge_hbm.at[pl.ds(ebase + j * C, C)], buf, sem)

    bufs = ((d0, ds0), (d1, ds1), (d2, ds2), (d3, ds3))
    for k in range(4):
        d_copy(k, *bufs[k]).start()

    def body(i, carry):
        for k in range(4):
            j = 4 * i + k
            d_copy(j, *bufs[k]).wait()
            pltpu.sync_copy(ones_v, acc.at[bufs[k][0]], add=True)
            d_copy(j + 4, *bufs[k]).start()
        return carry

    lax.fori_loop(0, CF // 4 - 1, body, 0)
    for k in range(4):
        j = (CF // 4 - 1) * 4 + k
        d_copy(j, *bufs[k]).wait()
        pltpu.sync_copy(ones_v, acc.at[bufs[k][0]], add=True)
        if j + 4 < CF:
            d_copy(j + 4, *bufs[k]).start()
    for k in range(CF % 4):
        j = (CF // 4) * 4 + k
        d_copy(j, *bufs[k]).wait()
        pltpu.sync_copy(ones_v, acc.at[bufs[k][0]], add=True)
    pltpu.sync_copy(edge_hbm.at[pl.ds(ebase + CF * C, TAIL)], dt)
    pltpu.sync_copy(ones_v.at[pl.ds(0, TAIL)], acc.at[dt], add=True)
    plsc.subcore_barrier()

    pltpu.sync_copy(acc.at[pl.ds(rb, RPT)], zv)

    @pl.when(cid == 0)
    def _():
        pltpu.sync_copy(zv, out0.at[pl.ds(rb, RPT)])

    @pl.when(cid == 1)
    def _():
        pltpu.sync_copy(zv, out1.at[pl.ds(rb, RPT)])


@functools.partial(
    pl.kernel,
    out_type=(jax.ShapeDtypeStruct((NPAD, D), jnp.float32),
              jax.ShapeDtypeStruct((NPAD, D), jnp.float32)),
    mesh=_mesh,
    scratch_types=[
        pltpu.VMEM((EPT,), jnp.int32),
        pltpu.VMEM((C,), jnp.int32),
        pltpu.VMEM((C,), jnp.int32),
        pltpu.VMEM((TAIL,), jnp.int32),
        pltpu.VMEM((C, D), jnp.float32),
        pltpu.VMEM((C, D), jnp.float32),
        pltpu.VMEM((16, D), jnp.float32),
        pltpu.VMEM_SHARED((NPAD, D), jnp.float32),
        pltpu.SemaphoreType.DMA,
        pltpu.SemaphoreType.DMA,
        pltpu.SemaphoreType.DMA,
        pltpu.SemaphoreType.DMA,
        pltpu.SemaphoreType.DMA,
        pltpu.SemaphoreType.DMA,
    ],
)
def _scat_call(h_hbm, edge_hbm, out0, out1,
               srcv, d0, d1, dt, b0, b1, zb, acc,
               gs0, gs1, ds0, ds1, ss0, ss1):
    cid = lax.axis_index("c")
    sid = lax.axis_index("s")
    wid = cid * NS + sid
    sbase = wid * EPT
    ebase = E + wid * EPT

    for i in range(16):
        for j in range(D // 16):
            zb[i, pl.ds(j * 16, 16)] = jnp.zeros((16,), jnp.float32)

    rb = sid * RPT

    pltpu.make_async_copy(
        edge_hbm.at[pl.ds(sbase, EPT)], srcv, ss0).start()

    def zcp(k, carry):
        pltpu.make_async_copy(
            zb, acc.at[pl.ds(rb + k * 16, 16)], ss1).start()
        return carry

    lax.fori_loop(0, RPT // 16, zcp, 0)

    def zdr(k, carry):
        pltpu.make_async_copy(zb, acc.at[pl.ds(rb, 16)], ss1).wait()
        return carry

    lax.fori_loop(0, RPT // 16, zdr, 0)
    pltpu.make_async_copy(edge_hbm.at[pl.ds(sbase, EPT)], srcv, ss0).wait()
    plsc.subcore_barrier()

    def g_copy(j, buf, sem):
        return pltpu.make_async_copy(
            h_hbm.at[srcv.at[pl.ds(j * C, C)]], buf, sem)

    def d_copy(j, buf, sem):
        return pltpu.make_async_copy(
            edge_hbm.at[pl.ds(ebase + j * C, C)], buf, sem)

    def s_start(buf, dbuf, sem):
        pltpu.async_copy(buf, acc.at[dbuf], sem, add=True)

    def s_wait(buf, dbuf, sem):
        pltpu.make_async_copy(buf, acc.at[dbuf], sem).wait()

    d_copy(0, d0, ds0).start()
    g_copy(0, b0, gs0).start()
    d_copy(1, d1, ds1).start()
    g_copy(1, b1, gs1).start()

    def body(i, carry):
        j0 = 2 * i
        j1 = j0 + 1
        g_copy(j0, b0, gs0).wait()
        d_copy(j0, d0, ds0).wait()
        s_start(b0, d0, ss0)
        g_copy(j1, b1, gs1).wait()
        d_copy(j1, d1, ds1).wait()
        s_start(b1, d1, ss1)
        s_wait(b0, d0, ss0)
        d_copy(j0 + 2, d0, ds0).start()
        g_copy(j0 + 2, b0, gs0).start()
        s_wait(b1, d1, ss1)
        d_copy(j1 + 2, d1, ds1).start()
        g_copy(j1 + 2, b1, gs1).start()
        return carry

    lax.fori_loop(0, CF // 2 - 1, body, 0)
    g_copy(CF - 2, b0, gs0).wait()
    d_copy(CF - 2, d0, ds0).wait()
    s_start(b0, d0, ss0)
    g_copy(CF - 1, b1, gs1).wait()
    d_copy(CF - 1, d1, ds1).wait()
    s_start(b1, d1, ss1)
    s_wait(b0, d0, ss0)
    s_wait(b1, d1, ss1)

    pltpu.sync_copy(edge_hbm.at[pl.ds(ebase + CF * C, TAIL)], dt)
    pltpu.make_async_copy(
        h_hbm.at[srcv.at[pl.ds(CF * C, TAIL)]],
        b0.at[pl.ds(0, TAIL)], gs0).start()
    pltpu.make_async_copy(
        h_hbm.at[srcv.at[pl.ds(CF * C, TAIL)]],
        b0.at[pl.ds(0, TAIL)], gs0).wait()
    pltpu.sync_copy(b0.at[pl.ds(0, TAIL)], acc.at[dt], add=True)
    plsc.subcore_barrier()

    def wb(out_ref):
        def rd(k, buf, sem):
            return pltpu.make_async_copy(
                acc.at[pl.ds(rb + k * C, C)], buf, sem)

        rd(0, b0, gs0).start()
        for k in range(RPT // C):
            buf, sem = (b0, gs0) if k % 2 == 0 else (b1, gs1)
            rd(k, buf, sem).wait()
            if k + 1 < RPT // C:
                nbuf, nsem = (b0, gs0) if (k + 1) % 2 == 0 else (b1, gs1)
                rd(k + 1, nbuf, nsem).start()
            pltpu.sync_copy(buf, out_ref.at[pl.ds(rb + k * C, C)])

    @pl.when(cid == 0)
    def _():
        wb(out0)

    @pl.when(cid == 1)
    def _():
        wb(out1)


_R = 1024


def _mm1_body(x_ref, w_ref, d0_ref, d1_ref, h_ref, dinv_ref):
    d = d0_ref[...] + d1_ref[...]
    dinv = jnp.where(d > 0, lax.rsqrt(jnp.where(d > 0, d, 1.0)), 0.0)
    h = jnp.dot(x_ref[...], w_ref[...], preferred_element_type=jnp.float32)
    h_ref[...] = h * dinv
    dinv_ref[...] = dinv


def _l2_body(p0_ref, p1_ref, dinv_ref, b1_ref, w_ref, out_ref):
    dinv = dinv_ref[...]
    h = jnp.maximum((p0_ref[...] + p1_ref[...]) * dinv + b1_ref[...], 0.0)
    out_ref[...] = jnp.dot(
        h, w_ref[...], preferred_element_type=jnp.float32) * dinv


def _comb_body(q0_ref, q1_ref, dinv_ref, b2_ref, out_ref):
    out_ref[...] = ((q0_ref[...] + q1_ref[...]) * dinv_ref[...]
                    + b2_ref[...])


def _row_spec(w):
    return pl.BlockSpec((_R, w), lambda i: (i, 0))


def _rep_spec(h, w):
    return pl.BlockSpec((h, w), lambda i: (0, 0))


_mm1 = pl.pallas_call(
    _mm1_body,
    grid=(NPAD // _R,),
    in_specs=[_row_spec(D), _rep_spec(D, D), _row_spec(1), _row_spec(1)],
    out_specs=[_row_spec(D), _row_spec(1)],
    out_shape=[jax.ShapeDtypeStruct((NPAD, D), jnp.float32),
               jax.ShapeDtypeStruct((NPAD, 1), jnp.float32)],
)

_l2 = pl.pallas_call(
    _l2_body,
    grid=(NPAD // _R,),
    in_specs=[_row_spec(D), _row_spec(D), _row_spec(1), _rep_spec(1, D),
              _rep_spec(D, D)],
    out_specs=_row_spec(D),
    out_shape=jax.ShapeDtypeStruct((NPAD, D), jnp.float32),
)

_comb = pl.pallas_call(
    _comb_body,
    grid=(NPAD // _R,),
    in_specs=[_row_spec(D), _row_spec(D), _row_spec(1), _rep_spec(1, D)],
    out_specs=_row_spec(D),
    out_shape=jax.ShapeDtypeStruct((N, D), jnp.float32),
)


def kernel(x, edge_index, W1, b1, W2, b2):
    edge_flat = edge_index.reshape(2 * E)

    g0, g1 = _deg_call(edge_flat)
    d0 = g0.reshape(NPAD, 1)
    d1 = g1.reshape(NPAD, 1)

    x_pad = jnp.pad(x, ((0, NPAD - N), (0, 0)))
    h1, dinv = _mm1(x_pad, W1, d0, d1)

    p0, p1 = _scat_call(h1, edge_flat)
    h2 = _l2(p0, p1, dinv, b1.reshape(1, D), W2)

    q0, q1 = _scat_call(h2, edge_flat)
    return _comb(q0, q1, dinv, b2.reshape(1, D))


# speedup vs baseline: 1.2611x; 1.2611x over previous
import functools

import jax
import jax.numpy as jnp
from jax import lax
from jax.experimental import pallas as pl
from jax.experimental.pallas import tpu as pltpu
from jax.experimental.pallas import tpu_sc as plsc

N = 10000
E = 320000
D = 128
NPAD = 10240

NC = 2
NS = 16
NW = NC * NS
C = 128
EPT = E // NW
CF = EPT // C
TAIL = EPT - CF * C
RPT = NPAD // NS

_mesh = plsc.VectorSubcoreMesh(
    core_axis_name="c", subcore_axis_name="s", num_cores=NC, num_subcores=NS)


@functools.partial(
    pl.kernel,
    out_type=(jax.ShapeDtypeStruct((NPAD,), jnp.float32),
              jax.ShapeDtypeStruct((NPAD,), jnp.float32)),
    mesh=_mesh,
    scratch_types=[
        pltpu.VMEM((C,), jnp.int32),
        pltpu.VMEM((C,), jnp.int32),
        pltpu.VMEM((C,), jnp.int32),
        pltpu.VMEM((C,), jnp.int32),
        pltpu.VMEM((TAIL,), jnp.int32),
        pltpu.VMEM((C,), jnp.float32),
        pltpu.VMEM((RPT,), jnp.float32),
        pltpu.VMEM_SHARED((NPAD,), jnp.float32),
        pltpu.SemaphoreType.DMA,
        pltpu.SemaphoreType.DMA,
        pltpu.SemaphoreType.DMA,
        pltpu.SemaphoreType.DMA,
    ],
)
def _deg_call(edge_hbm, out0, out1, d0, d1, d2, d3, dt, ones_v, zv, acc,
              ds0, ds1, ds2, ds3):
    cid = lax.axis_index("c")
    sid = lax.axis_index("s")
    wid = cid * NS + sid
    ebase = E + wid * EPT

    for k in range(RPT // 16):
        zv[pl.ds(k * 16, 16)] = jnp.zeros((16,), jnp.float32)
    for k in range(C // 16):
        ones_v[pl.ds(k * 16, 16)] = jnp.ones((16,), jnp.float32)

    rb = sid * RPT
    pltpu.sync_copy(zv, acc.at[pl.ds(rb, RPT)])
    plsc.subcore_barrier()

    def d_copy(j, buf, sem):
        return pltpu.make_async_copy(
            edge_hbm.at[pl.ds(ebase + j * C, C)], buf, sem)

    bufs = ((d0, ds0), (d1, ds1), (d2, ds2), (d3, ds3))
    for k in range(4):
        d_copy(k, *bufs[k]).start()

    def body(i, carry):
        for k in range(4):
            j = 4 * i + k
            d_copy(j, *bufs[k]).wait()
            pltpu.sync_copy(ones_v, acc.at[bufs[k][0]], add=True)
            d_copy(j + 4, *bufs[k]).start()
        return carry

    lax.fori_loop(0, CF // 4 - 1, body, 0)
    for k in range(4):
        j = (CF // 4 - 1) * 4 + k
        d_copy(j, *bufs[k]).wait()
        pltpu.sync_copy(ones_v, acc.at[bufs[k][0]], add=True)
        if j + 4 < CF:
            d_copy(j + 4, *bufs[k]).start()
    for k in range(CF % 4):
        j = (CF // 4) * 4 + k
        d_copy(j, *bufs[k]).wait()
        pltpu.sync_copy(ones_v, acc.at[bufs[k][0]], add=True)
    pltpu.sync_copy(edge_hbm.at[pl.ds(ebase + CF * C, TAIL)], dt)
    pltpu.sync_copy(ones_v.at[pl.ds(0, TAIL)], acc.at[dt], add=True)
    plsc.subcore_barrier()

    pltpu.sync_copy(acc.at[pl.ds(rb, RPT)], zv)

    @pl.when(cid == 0)
    def _():
        pltpu.sync_copy(zv, out0.at[pl.ds(rb, RPT)])

    @pl.when(cid == 1)
    def _():
        pltpu.sync_copy(zv, out1.at[pl.ds(rb, RPT)])


@functools.partial(
    pl.kernel,
    out_type=(jax.ShapeDtypeStruct((NPAD, D), jnp.float32),
              jax.ShapeDtypeStruct((NPAD, D), jnp.float32)),
    mesh=_mesh,
    scratch_types=[
        pltpu.VMEM((EPT,), jnp.int32),
        pltpu.VMEM((C,), jnp.int32),
        pltpu.VMEM((C,), jnp.int32),
        pltpu.VMEM((TAIL,), jnp.int32),
        pltpu.VMEM((C, D), jnp.float32),
        pltpu.VMEM((C, D), jnp.float32),
        pltpu.VMEM((16, D), jnp.float32),
        pltpu.VMEM_SHARED((NPAD, D), jnp.float32),
        pltpu.SemaphoreType.DMA,
        pltpu.SemaphoreType.DMA,
        pltpu.SemaphoreType.DMA,
        pltpu.SemaphoreType.DMA,
        pltpu.SemaphoreType.DMA,
        pltpu.SemaphoreType.DMA,
    ],
)
def _scat_call(h_hbm, edge_hbm, out0, out1,
               srcv, d0, d1, dt, b0, b1, zb, acc,
               gs0, gs1, ds0, ds1, ss0, ss1):
    cid = lax.axis_index("c")
    sid = lax.axis_index("s")
    wid = cid * NS + sid
    sbase = wid * EPT
    ebase = E + wid * EPT

    for i in range(16):
        for j in range(D // 16):
            zb[i, pl.ds(j * 16, 16)] = jnp.zeros((16,), jnp.float32)

    rb = sid * RPT

    pltpu.make_async_copy(
        edge_hbm.at[pl.ds(sbase, EPT)], srcv, ss0).start()

    def zcp(k, carry):
        pltpu.make_async_copy(
            zb, acc.at[pl.ds(rb + k * 16, 16)], ss1).start()
        return carry

    lax.fori_loop(0, RPT // 16, zcp, 0)

    def zdr(k, carry):
        pltpu.make_async_copy(zb, acc.at[pl.ds(rb, 16)], ss1).wait()
        return carry

    lax.fori_loop(0, RPT // 16, zdr, 0)
    pltpu.make_async_copy(edge_hbm.at[pl.ds(sbase, EPT)], srcv, ss0).wait()
    plsc.subcore_barrier()

    def g_copy(j, buf, sem):
        return pltpu.make_async_copy(
            h_hbm.at[srcv.at[pl.ds(j * C, C)]], buf, sem)

    def d_copy(j, buf, sem):
        return pltpu.make_async_copy(
            edge_hbm.at[pl.ds(ebase + j * C, C)], buf, sem)

    def s_start(buf, dbuf, sem):
        pltpu.async_copy(buf, acc.at[dbuf], sem, add=True)

    def s_wait(buf, dbuf, sem):
        pltpu.make_async_copy(buf, acc.at[dbuf], sem).wait()

    d_copy(0, d0, ds0).start()
    g_copy(0, b0, gs0).start()
    d_copy(1, d1, ds1).start()
    g_copy(1, b1, gs1).start()

    def body(i, carry):
        j0 = 2 * i
        j1 = j0 + 1
        g_copy(j0, b0, gs0).wait()
        d_copy(j0, d0, ds0).wait()
        pltpu.sync_copy(b0, acc.at[d0], add=True)
        d_copy(j0 + 2, d0, ds0).start()
        g_copy(j0 + 2, b0, gs0).start()
        g_copy(j1, b1, gs1).wait()
        d_copy(j1, d1, ds1).wait()
        pltpu.sync_copy(b1, acc.at[d1], add=True)
        d_copy(j1 + 2, d1, ds1).start()
        g_copy(j1 + 2, b1, gs1).start()
        return carry

    lax.fori_loop(0, CF // 2 - 1, body, 0)
    g_copy(CF - 2, b0, gs0).wait()
    d_copy(CF - 2, d0, ds0).wait()
    pltpu.sync_copy(b0, acc.at[d0], add=True)
    g_copy(CF - 1, b1, gs1).wait()
    d_copy(CF - 1, d1, ds1).wait()
    pltpu.sync_copy(b1, acc.at[d1], add=True)

    pltpu.sync_copy(edge_hbm.at[pl.ds(ebase + CF * C, TAIL)], dt)
    pltpu.make_async_copy(
        h_hbm.at[srcv.at[pl.ds(CF * C, TAIL)]],
        b0.at[pl.ds(0, TAIL)], gs0).start()
    pltpu.make_async_copy(
        h_hbm.at[srcv.at[pl.ds(CF * C, TAIL)]],
        b0.at[pl.ds(0, TAIL)], gs0).wait()
    pltpu.sync_copy(b0.at[pl.ds(0, TAIL)], acc.at[dt], add=True)
    plsc.subcore_barrier()

    def wb(out_ref):
        def rd(k, buf, sem):
            return pltpu.make_async_copy(
                acc.at[pl.ds(rb + k * C, C)], buf, sem)

        rd(0, b0, gs0).start()
        for k in range(RPT // C):
            buf, sem = (b0, gs0) if k % 2 == 0 else (b1, gs1)
            rd(k, buf, sem).wait()
            if k + 1 < RPT // C:
                nbuf, nsem = (b0, gs0) if (k + 1) % 2 == 0 else (b1, gs1)
                rd(k + 1, nbuf, nsem).start()
            pltpu.sync_copy(buf, out_ref.at[pl.ds(rb + k * C, C)])

    @pl.when(cid == 0)
    def _():
        wb(out0)

    @pl.when(cid == 1)
    def _():
        wb(out1)


_R = 1024


def _mm1_body(x_ref, w_ref, d0_ref, d1_ref, h_ref, dinv_ref):
    d = d0_ref[...] + d1_ref[...]
    dinv = jnp.where(d > 0, lax.rsqrt(jnp.where(d > 0, d, 1.0)), 0.0)
    h = jnp.dot(x_ref[...], w_ref[...], preferred_element_type=jnp.float32)
    h_ref[...] = h * dinv
    dinv_ref[...] = dinv


def _l2_body(p0_ref, p1_ref, dinv_ref, b1_ref, w_ref, out_ref):
    dinv = dinv_ref[...]
    h = jnp.maximum((p0_ref[...] + p1_ref[...]) * dinv + b1_ref[...], 0.0)
    out_ref[...] = jnp.dot(
        h, w_ref[...], preferred_element_type=jnp.float32) * dinv


def _comb_body(q0_ref, q1_ref, dinv_ref, b2_ref, out_ref):
    out_ref[...] = ((q0_ref[...] + q1_ref[...]) * dinv_ref[...]
                    + b2_ref[...])


def _row_spec(w):
    return pl.BlockSpec((_R, w), lambda i: (i, 0))


def _rep_spec(h, w):
    return pl.BlockSpec((h, w), lambda i: (0, 0))


_mm1 = pl.pallas_call(
    _mm1_body,
    grid=(NPAD // _R,),
    in_specs=[_row_spec(D), _rep_spec(D, D), _row_spec(1), _row_spec(1)],
    out_specs=[_row_spec(D), _row_spec(1)],
    out_shape=[jax.ShapeDtypeStruct((NPAD, D), jnp.float32),
               jax.ShapeDtypeStruct((NPAD, 1), jnp.float32)],
)

_l2 = pl.pallas_call(
    _l2_body,
    grid=(NPAD // _R,),
    in_specs=[_row_spec(D), _row_spec(D), _row_spec(1), _rep_spec(1, D),
              _rep_spec(D, D)],
    out_specs=_row_spec(D),
    out_shape=jax.ShapeDtypeStruct((NPAD, D), jnp.float32),
)

_comb = pl.pallas_call(
    _comb_body,
    grid=(NPAD // _R,),
    in_specs=[_row_spec(D), _row_spec(D), _row_spec(1), _rep_spec(1, D)],
    out_specs=_row_spec(D),
    out_shape=jax.ShapeDtypeStruct((N, D), jnp.float32),
)


def kernel(x, edge_index, W1, b1, W2, b2):
    edge_flat = edge_index.reshape(2 * E)

    g0, g1 = _deg_call(edge_flat)
    d0 = g0.reshape(NPAD, 1)
    d1 = g1.reshape(NPAD, 1)

    x_pad = jnp.pad(x, ((0, NPAD - N), (0, 0)))
    h1, dinv = _mm1(x_pad, W1, d0, d1)

    p0, p1 = _scat_call(h1, edge_flat)
    h2 = _l2(p0, p1, dinv, b1.reshape(1, D), W2)

    q0, q1 = _scat_call(h2, edge_flat)
    return _comb(q0, q1, dinv, b2.reshape(1, D))


# speedup vs baseline: 1.2935x; 1.0256x over previous
import functools

import jax
import jax.numpy as jnp
from jax import lax
from jax.experimental import pallas as pl
from jax.experimental.pallas import tpu as pltpu
from jax.experimental.pallas import tpu_sc as plsc

N = 10000
E = 320000
D = 128
NPAD = 10240

NC = 2
NS = 16
NW = NC * NS
C = 128
EPT = E // NW
CF = EPT // C
TAIL = EPT - CF * C
RPT = NPAD // NS

_mesh = plsc.VectorSubcoreMesh(
    core_axis_name="c", subcore_axis_name="s", num_cores=NC, num_subcores=NS)


@functools.partial(
    pl.kernel,
    out_type=(jax.ShapeDtypeStruct((NPAD,), jnp.float32),
              jax.ShapeDtypeStruct((NPAD,), jnp.float32)),
    mesh=_mesh,
    scratch_types=[
        pltpu.VMEM((C,), jnp.int32),
        pltpu.VMEM((C,), jnp.int32),
        pltpu.VMEM((C,), jnp.int32),
        pltpu.VMEM((C,), jnp.int32),
        pltpu.VMEM((TAIL,), jnp.int32),
        pltpu.VMEM((C,), jnp.float32),
        pltpu.VMEM((RPT,), jnp.float32),
        pltpu.VMEM_SHARED((NPAD,), jnp.float32),
        pltpu.SemaphoreType.DMA,
        pltpu.SemaphoreType.DMA,
        pltpu.SemaphoreType.DMA,
        pltpu.SemaphoreType.DMA,
    ],
)
def _deg_call(edge_hbm, out0, out1, d0, d1, d2, d3, dt, ones_v, zv, acc,
              ds0, ds1, ds2, ds3):
    cid = lax.axis_index("c")
    sid = lax.axis_index("s")
    wid = cid * NS + sid
    ebase = E + wid * EPT

    for k in range(RPT // 16):
        zv[pl.ds(k * 16, 16)] = jnp.zeros((16,), jnp.float32)
    for k in range(C // 16):
        ones_v[pl.ds(k * 16, 16)] = jnp.ones((16,), jnp.float32)

    rb = sid * RPT
    pltpu.sync_copy(zv, acc.at[pl.ds(rb, RPT)])
    plsc.subcore_barrier()

    def d_copy(j, buf, sem):
        return pltpu.make_async_copy(
            edge_hbm.at[pl.ds(ebase + j * C, C)], buf, sem)

    bufs = ((d0, ds0), (d1, ds1), (d2, ds2), (d3, ds3))
    for k in range(4):
        d_copy(k, *bufs[k]).start()

    def body(i, carry):
        for k in range(4):
            j = 4 * i + k
            d_copy(j, *bufs[k]).wait()
            pltpu.sync_copy(ones_v, acc.at[bufs[k][0]], add=True)
            d_copy(j + 4, *bufs[k]).start()
        return carry

    lax.fori_loop(0, CF // 4 - 1, body, 0)
    for k in range(4):
        j = (CF // 4 - 1) * 4 + k
        d_copy(j, *bufs[k]).wait()
        pltpu.sync_copy(ones_v, acc.at[bufs[k][0]], add=True)
        if j + 4 < CF:
            d_copy(j + 4, *bufs[k]).start()
    for k in range(CF % 4):
        j = (CF // 4) * 4 + k
        d_copy(j, *bufs[k]).wait()
        pltpu.sync_copy(ones_v, acc.at[bufs[k][0]], add=True)
    pltpu.sync_copy(edge_hbm.at[pl.ds(ebase + CF * C, TAIL)], dt)
    pltpu.sync_copy(ones_v.at[pl.ds(0, TAIL)], acc.at[dt], add=True)
    plsc.subcore_barrier()

    pltpu.sync_copy(acc.at[pl.ds(rb, RPT)], zv)

    @pl.when(cid == 0)
    def _():
        pltpu.sync_copy(zv, out0.at[pl.ds(rb, RPT)])

    @pl.when(cid == 1)
    def _():
        pltpu.sync_copy(zv, out1.at[pl.ds(rb, RPT)])


@functools.partial(
    pl.kernel,
    out_type=(jax.ShapeDtypeStruct((NPAD, D), jnp.float32),
              jax.ShapeDtypeStruct((NPAD, D), jnp.float32)),
    mesh=_mesh,
    scratch_types=[
        pltpu.VMEM((EPT,), jnp.int32),
        pltpu.VMEM((C,), jnp.int32),
        pltpu.VMEM((C,), jnp.int32),
        pltpu.VMEM((TAIL,), jnp.int32),
        pltpu.VMEM((C, D), jnp.float32),
        pltpu.VMEM((C, D), jnp.float32),
        pltpu.VMEM((16, D), jnp.float32),
        pltpu.VMEM_SHARED((NPAD, D), jnp.float32),
        pltpu.SemaphoreType.DMA,
        pltpu.SemaphoreType.DMA,
        pltpu.SemaphoreType.DMA,
        pltpu.SemaphoreType.DMA,
        pltpu.SemaphoreType.DMA,
        pltpu.SemaphoreType.DMA,
    ],
)
def _scat_call(h_hbm, edge_hbm, out0, out1,
               srcv, d0, d1, dt, b0, b1, zb, acc,
               gs0, gs1, ds0, ds1, ss0, ss1):
    cid = lax.axis_index("c")
    sid = lax.axis_index("s")
    wid = cid * NS + sid
    sbase = wid * EPT
    ebase = E + wid * EPT

    for i in range(16):
        for j in range(D // 16):
            zb[i, pl.ds(j * 16, 16)] = jnp.zeros((16,), jnp.float32)

    rb = sid * RPT

    pltpu.make_async_copy(
        edge_hbm.at[pl.ds(sbase, EPT)], srcv, ss0).start()

    def zcp(k, carry):
        pltpu.make_async_copy(
            zb, acc.at[pl.ds(rb + k * 16, 16)], ss1).start()
        return carry

    lax.fori_loop(0, RPT // 16, zcp, 0)

    def zdr(k, carry):
        pltpu.make_async_copy(zb, acc.at[pl.ds(rb, 16)], ss1).wait()
        return carry

    lax.fori_loop(0, RPT // 16, zdr, 0)
    pltpu.make_async_copy(edge_hbm.at[pl.ds(sbase, EPT)], srcv, ss0).wait()
    plsc.subcore_barrier()

    def g_copy(j, buf, sem):
        return pltpu.make_async_copy(
            h_hbm.at[srcv.at[pl.ds(j * C, C)]], buf, sem)

    def d_copy(j, buf, sem):
        return pltpu.make_async_copy(
            edge_hbm.at[pl.ds(ebase + j * C, C)], buf, sem)

    def s_start(buf, dbuf, sem):
        pltpu.async_copy(buf, acc.at[dbuf], sem, add=True)

    def s_wait(buf, dbuf, sem):
        pltpu.make_async_copy(buf, acc.at[dbuf], sem).wait()

    d_copy(0, d0, ds0).start()
    g_copy(0, b0, gs0).start()
    d_copy(1, d1, ds1).start()
    g_copy(1, b1, gs1).start()

    def body(i, carry):
        j0 = 2 * i
        j1 = j0 + 1
        g_copy(j0, b0, gs0).wait()
        d_copy(j0, d0, ds0).wait()
        pltpu.sync_copy(b0, acc.at[d0], add=True)
        d_copy(j0 + 2, d0, ds0).start()
        g_copy(j0 + 2, b0, gs0).start()
        g_copy(j1, b1, gs1).wait()
        d_copy(j1, d1, ds1).wait()
        pltpu.sync_copy(b1, acc.at[d1], add=True)
        d_copy(j1 + 2, d1, ds1).start()
        g_copy(j1 + 2, b1, gs1).start()
        return carry

    lax.fori_loop(0, CF // 2 - 1, body, 0)
    g_copy(CF - 2, b0, gs0).wait()
    d_copy(CF - 2, d0, ds0).wait()
    pltpu.sync_copy(b0, acc.at[d0], add=True)
    g_copy(CF - 1, b1, gs1).wait()
    d_copy(CF - 1, d1, ds1).wait()
    pltpu.sync_copy(b1, acc.at[d1], add=True)

    pltpu.sync_copy(edge_hbm.at[pl.ds(ebase + CF * C, TAIL)], dt)
    pltpu.make_async_copy(
        h_hbm.at[srcv.at[pl.ds(CF * C, TAIL)]],
        b0.at[pl.ds(0, TAIL)], gs0).start()
    pltpu.make_async_copy(
        h_hbm.at[srcv.at[pl.ds(CF * C, TAIL)]],
        b0.at[pl.ds(0, TAIL)], gs0).wait()
    pltpu.sync_copy(b0.at[pl.ds(0, TAIL)], acc.at[dt], add=True)
    plsc.subcore_barrier()

    def wb(out_ref):
        def rd(k, buf, sem):
            return pltpu.make_async_copy(
                acc.at[pl.ds(rb + k * C, C)], buf, sem)

        rd(0, b0, gs0).start()
        for k in range(RPT // C):
            buf, sem = (b0, gs0) if k % 2 == 0 else (b1, gs1)
            rd(k, buf, sem).wait()
            if k + 1 < RPT // C:
                nbuf, nsem = (b0, gs0) if (k + 1) % 2 == 0 else (b1, gs1)
                rd(k + 1, nbuf, nsem).start()
            pltpu.sync_copy(buf, out_ref.at[pl.ds(rb + k * C, C)])

    @pl.when(cid == 0)
    def _():
        wb(out0)

    @pl.when(cid == 1)
    def _():
        wb(out1)


_R = 1024


def _mm_body(x_ref, w_ref, h_ref):
    h_ref[...] = jnp.dot(
        x_ref[...], w_ref[...], preferred_element_type=jnp.float32)


def _scale_body(h_ref, g0_ref, g1_ref, h1_ref, dinv_ref):
    d = g0_ref[...] + g1_ref[...]
    dv = jnp.where(d > 0, lax.rsqrt(jnp.where(d > 0, d, 1.0)), 0.0)
    dinv = dv[:, None]
    h1_ref[...] = h_ref[...] * dinv
    dinv_ref[...] = dinv


def _l2_body(p0_ref, p1_ref, dinv_ref, b1_ref, w_ref, out_ref):
    dinv = dinv_ref[...]
    h = jnp.maximum((p0_ref[...] + p1_ref[...]) * dinv + b1_ref[...], 0.0)
    out_ref[...] = jnp.dot(
        h, w_ref[...], preferred_element_type=jnp.float32) * dinv


def _comb_body(q0_ref, q1_ref, dinv_ref, b2_ref, out_ref):
    out_ref[...] = ((q0_ref[...] + q1_ref[...]) * dinv_ref[...]
                    + b2_ref[...])


def _row_spec(w):
    return pl.BlockSpec((_R, w), lambda i: (i, 0))


def _rep_spec(h, w):
    return pl.BlockSpec((h, w), lambda i: (0, 0))


_mm = pl.pallas_call(
    _mm_body,
    grid=(NPAD // _R,),
    in_specs=[_row_spec(D), _rep_spec(D, D)],
    out_specs=_row_spec(D),
    out_shape=jax.ShapeDtypeStruct((NPAD, D), jnp.float32),
)

_scale = pl.pallas_call(
    _scale_body,
    grid=(NPAD // _R,),
    in_specs=[_row_spec(D),
              pl.BlockSpec((_R,), lambda i: (i,)),
              pl.BlockSpec((_R,), lambda i: (i,))],
    out_specs=[_row_spec(D), _row_spec(1)],
    out_shape=[jax.ShapeDtypeStruct((NPAD, D), jnp.float32),
               jax.ShapeDtypeStruct((NPAD, 1), jnp.float32)],
)

_l2 = pl.pallas_call(
    _l2_body,
    grid=(NPAD // _R,),
    in_specs=[_row_spec(D), _row_spec(D), _row_spec(1), _rep_spec(1, D),
              _rep_spec(D, D)],
    out_specs=_row_spec(D),
    out_shape=jax.ShapeDtypeStruct((NPAD, D), jnp.float32),
)

_comb = pl.pallas_call(
    _comb_body,
    grid=(NPAD // _R,),
    in_specs=[_row_spec(D), _row_spec(D), _row_spec(1), _rep_spec(1, D)],
    out_specs=_row_spec(D),
    out_shape=jax.ShapeDtypeStruct((N, D), jnp.float32),
)


def kernel(x, edge_index, W1, b1, W2, b2):
    edge_flat = edge_index.reshape(2 * E)

    g0, g1 = _deg_call(edge_flat)

    x_pad = jnp.pad(x, ((0, NPAD - N), (0, 0)))
    h0 = _mm(x_pad, W1)
    h1, dinv = _scale(h0, g0, g1)

    p0, p1 = _scat_call(h1, edge_flat)
    h2 = _l2(p0, p1, dinv, b1.reshape(1, D), W2)

    q0, q1 = _scat_call(h2, edge_flat)
    return _comb(q0, q1, dinv, b2.reshape(1, D))
